# Initial kernel scaffold; baseline (speedup 1.0000x reference)
import jax
import jax.numpy as jnp
from jax import lax
from jax.experimental import pallas as pl
from jax.experimental.pallas import tpu as pltpu
from jax.experimental.pallas import tpu_sc as plsc

_N = 100000
_H = 128
_NB = 100000
_NC = 100000

_B = 128
_NW = 32
_NPAD = 102400
_CPW = (_NPAD // _B) // _NW
_G = _B // 16


def _sc_body(feat_hbm, btab_hbm, ctab_hbm, wb_hbm, wc_hbm, out_hbm,
             slab_v, idx0_v, idx1_v, idxc_v, sb_v, sc_v, wb_v, wc_v, out_v,
             sem_in, sem_g):
    wid = lax.axis_index("s") * 2 + lax.axis_index("c")
    base0 = wid * (_CPW * _B)

    pltpu.sync_copy(wb_hbm, wb_v)
    pltpu.sync_copy(wc_hbm, wc_v)
    wbs = [wb_v[pl.ds(h * 16, 16)] for h in range(_H // 16)]
    wcs = [wc_v[pl.ds(h * 16, 16)] for h in range(_H // 16)]

    def chunk(j, carry):
        base = base0 + j * _B
        pltpu.async_copy(feat_hbm.at[pl.ds(base, _B)], slab_v, sem_in).wait()

        for g in range(_G):
            rows = lax.iota(jnp.int32, 16) + g * 16

            def col(c):
                cc = jnp.full((16,), c, jnp.int32)
                return plsc.load_gather(slab_v, [rows, cc])

            i0 = jnp.clip((col(0) * _NB).astype(jnp.int32), 0, _NB - 1)
            i1 = jnp.clip((col(1) * _NB).astype(jnp.int32), 0, _NB - 1)
            ic = jnp.clip((col(11) * _NC).astype(jnp.int32), 0, _NC - 1)
            idx0_v[pl.ds(g * 16, 16)] = i0
            idx1_v[pl.ds(g * 16, 16)] = i1
            idxc_v[pl.ds(g * 16, 16)] = ic

            s = col(2)
            for c in range(3, 10):
                s = s + col(c)
            sb_v[pl.ds(g * 16, 16)] = s
            sc_v[pl.ds(g * 16, 16)] = col(10)

        def row(r, _):
            vb = plsc.load_gather(sb_v, [jnp.full((16,), 0, jnp.int32) + r])
            vc = plsc.load_gather(sc_v, [jnp.full((16,), 0, jnp.int32) + r])
            for h in range(_H // 16):
                out_v[r, pl.ds(h * 16, 16)] = vb * wbs[h] + vc * wcs[h]
            return 0

        lax.fori_loop(0, _B, row, 0)

        cp0 = pltpu.async_copy(btab_hbm.at[idx0_v], out_v, sem_g, add=True)
        cp1 = pltpu.async_copy(btab_hbm.at[idx1_v], out_v, sem_g, add=True)
        cp2 = pltpu.async_copy(ctab_hbm.at[idxc_v], out_v, sem_g, add=True)
        cp0.wait()
        cp1.wait()
        cp2.wait()

        pltpu.sync_copy(out_v, out_hbm.at[pl.ds(base, _B)])
        return carry

    lax.fori_loop(0, _CPW, chunk, 0)


def kernel(node_feat, basic_table, contact_table, W_basic, W_contact):
    feat = jnp.pad(node_feat, ((0, _NPAD - _N), (0, 0)))
    wb = W_basic.reshape(_H)
    wc = W_contact.reshape(_H)
    run = pl.kernel(
        _sc_body,
        out_type=jax.ShapeDtypeStruct((_NPAD, _H), jnp.float32),
        mesh=plsc.VectorSubcoreMesh(core_axis_name="c", subcore_axis_name="s"),
        scratch_types=[
            pltpu.VMEM((_B, 12), jnp.float32),
            pltpu.VMEM((_B,), jnp.int32),
            pltpu.VMEM((_B,), jnp.int32),
            pltpu.VMEM((_B,), jnp.int32),
            pltpu.VMEM((_B,), jnp.float32),
            pltpu.VMEM((_B,), jnp.float32),
            pltpu.VMEM((_H,), jnp.float32),
            pltpu.VMEM((_H,), jnp.float32),
            pltpu.VMEM((_B, _H), jnp.float32),
            pltpu.SemaphoreType.DMA,
            pltpu.SemaphoreType.DMA,
        ],
    )
    out = run(feat, basic_table, contact_table, wb, wc)
    return out[:_N].reshape(1, _N, _H)


# speedup vs baseline: 2.5795x; 2.5795x over previous
import jax
import jax.numpy as jnp
from jax import lax
from jax.experimental import pallas as pl
from jax.experimental.pallas import tpu as pltpu
from jax.experimental.pallas import tpu_sc as plsc

_N = 100000
_H = 128
_NB = 100000
_NC = 100000

_B = 128
_NW = 32
_NPAD = 102400
_CPW = (_NPAD // _B) // _NW
_G = _B // 16


def _sc_body(feat_hbm, btab_hbm, ctab_hbm, wb_hbm, wc_hbm, out_hbm,
             slab_v, idx0_v, idx1_v, idxc_v, sb_v, sc_v, wb_v, wc_v, out_v,
             sem_in, sem_g):
    wid = lax.axis_index("s") * 2 + lax.axis_index("c")
    base0 = wid * (_CPW * _B)

    pltpu.sync_copy(wb_hbm, wb_v)
    pltpu.sync_copy(wc_hbm, wc_v)
    wbs = [wb_v[pl.ds(h * 16, 16)] for h in range(_H // 16)]
    wcs = [wc_v[pl.ds(h * 16, 16)] for h in range(_H // 16)]

    def chunk(j, carry):
        base = base0 + j * _B
        pltpu.async_copy(feat_hbm.at[pl.ds(base * 12, _B * 12)], slab_v,
                         sem_in).wait()

        for g in range(_G):
            rows12 = (lax.iota(jnp.int32, 16) + g * 16) * 12

            def col(c):
                return plsc.load_gather(slab_v, [rows12 + c])

            i0 = jnp.clip((col(0) * _NB).astype(jnp.int32), 0, _NB - 1)
            i1 = jnp.clip((col(1) * _NB).astype(jnp.int32), 0, _NB - 1)
            ic = jnp.clip((col(11) * _NC).astype(jnp.int32), 0, _NC - 1)
            idx0_v[pl.ds(g * 16, 16)] = i0
            idx1_v[pl.ds(g * 16, 16)] = i1
            idxc_v[pl.ds(g * 16, 16)] = ic

            s = col(2)
            for c in range(3, 10):
                s = s + col(c)
            sb_v[pl.ds(g * 16, 16)] = s
            sc_v[pl.ds(g * 16, 16)] = col(10)

        def row(r, _):
            vb = plsc.load_gather(sb_v, [jnp.full((16,), 0, jnp.int32) + r])
            vc = plsc.load_gather(sc_v, [jnp.full((16,), 0, jnp.int32) + r])
            for h in range(_H // 16):
                out_v[r, pl.ds(h * 16, 16)] = vb * wbs[h] + vc * wcs[h]
            return 0

        lax.fori_loop(0, _B, row, 0)

        cp0 = pltpu.async_copy(btab_hbm.at[idx0_v], out_v, sem_g, add=True)
        cp1 = pltpu.async_copy(btab_hbm.at[idx1_v], out_v, sem_g, add=True)
        cp2 = pltpu.async_copy(ctab_hbm.at[idxc_v], out_v, sem_g, add=True)
        cp0.wait()
        cp1.wait()
        cp2.wait()

        pltpu.sync_copy(out_v, out_hbm.at[pl.ds(base, _B)])
        return carry

    lax.fori_loop(0, _CPW, chunk, 0)


def kernel(node_feat, basic_table, contact_table, W_basic, W_contact):
    feat = jnp.pad(node_feat, ((0, _NPAD - _N), (0, 0))).reshape(-1)
    wb = W_basic.reshape(_H)
    wc = W_contact.reshape(_H)
    run = pl.kernel(
        _sc_body,
        out_type=jax.ShapeDtypeStruct((_NPAD, _H), jnp.float32),
        mesh=plsc.VectorSubcoreMesh(core_axis_name="c", subcore_axis_name="s"),
        compiler_params=pltpu.CompilerParams(needs_layout_passes=False),
        scratch_types=[
            pltpu.VMEM((_B * 12,), jnp.float32),
            pltpu.VMEM((_B,), jnp.int32),
            pltpu.VMEM((_B,), jnp.int32),
            pltpu.VMEM((_B,), jnp.int32),
            pltpu.VMEM((_B,), jnp.float32),
            pltpu.VMEM((_B,), jnp.float32),
            pltpu.VMEM((_H,), jnp.float32),
            pltpu.VMEM((_H,), jnp.float32),
            pltpu.VMEM((_B, _H), jnp.float32),
            pltpu.SemaphoreType.DMA,
            pltpu.SemaphoreType.DMA,
        ],
    )
    out = run(feat, basic_table, contact_table, wb, wc)
    return out[:_N].reshape(1, _N, _H)


# speedup vs baseline: 6.8309x; 2.6481x over previous
import jax
import jax.numpy as jnp
from jax import lax
from jax.experimental import pallas as pl
from jax.experimental.pallas import tpu as pltpu
from jax.experimental.pallas import tpu_sc as plsc

_N = 100000
_H = 128
_NB = 100000
_NC = 100000

_B = 128
_RPW = 3200
_CPW = _RPW // _B
_LAST_FULL = 6
_TAIL = 32
_TAIL_BASE = 31 * _RPW + _LAST_FULL * _B


def _sc_body(feat_hbm, btab_hbm, ctab_hbm, wb_hbm, wc_hbm, out_hbm,
             slab_v, idx0_v, idx1_v, idxc_v, sb_v, sc_v, wb_v, wc_v, out_v,
             sem_in, sem_g):
    wid = lax.axis_index("s") * 2 + lax.axis_index("c")

    pltpu.sync_copy(wb_hbm, wb_v)
    pltpu.sync_copy(wc_hbm, wc_v)
    wbs = [wb_v[pl.ds(h * 16, 16)] for h in range(_H // 16)]
    wcs = [wc_v[pl.ds(h * 16, 16)] for h in range(_H // 16)]

    def do_chunk(base, nrows):
        pltpu.async_copy(feat_hbm.at[pl.ds(base, nrows)],
                         slab_v.at[pl.ds(0, nrows)], sem_in).wait()

        for g in range(nrows // 16):
            rows = lax.iota(jnp.int32, 16) + g * 16

            def col(c):
                cc = jnp.full((16,), c, jnp.int32)
                return plsc.load_gather(slab_v, [rows, cc])

            i0 = jnp.clip((col(0) * _NB).astype(jnp.int32), 0, _NB - 1)
            i1 = jnp.clip((col(1) * _NB).astype(jnp.int32), 0, _NB - 1)
            ic = jnp.clip((col(11) * _NC).astype(jnp.int32), 0, _NC - 1)
            idx0_v[pl.ds(g * 16, 16)] = i0
            idx1_v[pl.ds(g * 16, 16)] = i1
            idxc_v[pl.ds(g * 16, 16)] = ic

            s = col(2)
            for c in range(3, 10):
                s = s + col(c)
            sb_v[pl.ds(g * 16, 16)] = s
            sc_v[pl.ds(g * 16, 16)] = col(10)

        def row(r, _):
            vb = plsc.load_gather(sb_v, [jnp.full((16,), 0, jnp.int32) + r])
            vc = plsc.load_gather(sc_v, [jnp.full((16,), 0, jnp.int32) + r])
            for h in range(_H // 16):
                out_v[r, pl.ds(h * 16, 16)] = vb * wbs[h] + vc * wcs[h]
            return 0

        lax.fori_loop(0, nrows, row, 0)

        dst = out_v.at[pl.ds(0, nrows)]
        cp0 = pltpu.async_copy(btab_hbm.at[idx0_v.at[pl.ds(0, nrows)]],
                               dst, sem_g, add=True)
        cp1 = pltpu.async_copy(btab_hbm.at[idx1_v.at[pl.ds(0, nrows)]],
                               dst, sem_g, add=True)
        cp2 = pltpu.async_copy(ctab_hbm.at[idxc_v.at[pl.ds(0, nrows)]],
                               dst, sem_g, add=True)
        cp0.wait()
        cp1.wait()
        cp2.wait()

        pltpu.sync_copy(out_v.at[pl.ds(0, nrows)],
                        out_hbm.at[0, pl.ds(base, nrows)])

    nfull = jnp.where(wid == 31, _LAST_FULL, _CPW)

    def chunk(j, carry):
        do_chunk(wid * _RPW + j * _B, _B)
        return carry

    lax.fori_loop(0, nfull, chunk, 0)

    @pl.when(wid == 31)
    def _tail():
        do_chunk(jnp.int32(_TAIL_BASE), _TAIL)


def kernel(node_feat, basic_table, contact_table, W_basic, W_contact):
    wb = W_basic.reshape(_H)
    wc = W_contact.reshape(_H)
    run = pl.kernel(
        _sc_body,
        out_type=jax.ShapeDtypeStruct((1, _N, _H), jnp.float32),
        mesh=plsc.VectorSubcoreMesh(core_axis_name="c", subcore_axis_name="s"),
        compiler_params=pltpu.CompilerParams(needs_layout_passes=False),
        scratch_types=[
            pltpu.VMEM((_B, 12), jnp.float32),
            pltpu.VMEM((_B,), jnp.int32),
            pltpu.VMEM((_B,), jnp.int32),
            pltpu.VMEM((_B,), jnp.int32),
            pltpu.VMEM((_B,), jnp.float32),
            pltpu.VMEM((_B,), jnp.float32),
            pltpu.VMEM((_H,), jnp.float32),
            pltpu.VMEM((_H,), jnp.float32),
            pltpu.VMEM((_B, _H), jnp.float32),
            pltpu.SemaphoreType.DMA,
            pltpu.SemaphoreType.DMA,
        ],
    )
    return run(node_feat, basic_table, contact_table, wb, wc)


# speedup vs baseline: 9.8057x; 1.4355x over previous
import jax
import jax.numpy as jnp
from jax import lax
from jax.experimental import pallas as pl
from jax.experimental.pallas import tpu as pltpu
from jax.experimental.pallas import tpu_sc as plsc

_N = 100000
_H = 128
_NB = 100000
_NC = 100000

_B = 128
_HB = _H // 16
_RPW = 3200
_CPW = _RPW // _B
_LAST_FULL = 6
_TAIL = 32
_TAIL_BASE = 31 * _RPW + _LAST_FULL * _B


def _sc_body(feat_hbm, btab_hbm, ctab_hbm, wb_hbm, wc_hbm, out_hbm,
             slab_a, slab_b, idx0_a, idx0_b, idx1_a, idx1_b,
             idxc_a, idxc_b, sb_a, sb_b, sc_a, sc_b,
             wb_v, wc_v, out_a, out_b,
             sem_in_a, sem_in_b, sem_g_a, sem_g_b, sem_out_a, sem_out_b):
    wid = lax.axis_index("s") * 2 + lax.axis_index("c")
    row0 = wid * _RPW

    pltpu.sync_copy(wb_hbm, wb_v)
    pltpu.sync_copy(wc_hbm, wc_v)
    wbs = [wb_v[pl.ds(h * 16, 16)] for h in range(_HB)]
    wcs = [wc_v[pl.ds(h * 16, 16)] for h in range(_HB)]

    bufs = (
        (slab_a, idx0_a, idx1_a, idxc_a, sb_a, sc_a, out_a,
         sem_in_a, sem_g_a, sem_out_a),
        (slab_b, idx0_b, idx1_b, idxc_b, sb_b, sc_b, out_b,
         sem_in_b, sem_g_b, sem_out_b),
    )

    def fire_slab(j, p):
        slab, sem = bufs[p][0], bufs[p][7]
        pltpu.async_copy(feat_hbm.at[pl.ds(row0 + j * _B, _B)], slab, sem)

    def wait_slab(p):
        slab, sem = bufs[p][0], bufs[p][7]
        pltpu.make_async_copy(feat_hbm.at[pl.ds(0, _B)], slab, sem).wait()

    def compute(p, nrows=_B):
        slab, idx0, idx1, idxc, sb, sc = bufs[p][:6]
        for g in range(nrows // 16):
            rows = lax.iota(jnp.int32, 16) + g * 16

            def col(c):
                cc = jnp.full((16,), c, jnp.int32)
                return plsc.load_gather(slab, [rows, cc])

            idx0[pl.ds(g * 16, 16)] = jnp.clip(
                (col(0) * _NB).astype(jnp.int32), 0, _NB - 1)
            idx1[pl.ds(g * 16, 16)] = jnp.clip(
                (col(1) * _NB).astype(jnp.int32), 0, _NB - 1)
            idxc[pl.ds(g * 16, 16)] = jnp.clip(
                (col(11) * _NC).astype(jnp.int32), 0, _NC - 1)

            s = col(2)
            for c in range(3, 10):
                s = s + col(c)
            sb[pl.ds(g * 16, 16)] = s
            sc[pl.ds(g * 16, 16)] = col(10)

    def dense_init(p, nrows=_B):
        sb, sc, out = bufs[p][4], bufs[p][5], bufs[p][6]

        def one_row(r):
            rr = jnp.full((16,), 0, jnp.int32) + r
            vb = plsc.load_gather(sb, [rr])
            vc = plsc.load_gather(sc, [rr])
            for h in range(_HB):
                out[r, pl.ds(h * 16, 16)] = vb * wbs[h] + vc * wcs[h]

        def rows2(r, _):
            one_row(2 * r)
            one_row(2 * r + 1)
            return 0

        lax.fori_loop(0, nrows // 2, rows2, 0)

    def fire_gathers(p, nrows=_B):
        idx0, idx1, idxc, out, sem = (bufs[p][1], bufs[p][2], bufs[p][3],
                                      bufs[p][6], bufs[p][8])
        dst = out.at[pl.ds(0, nrows)]
        return (
            pltpu.async_copy(btab_hbm.at[idx0.at[pl.ds(0, nrows)]], dst, sem,
                             add=True),
            pltpu.async_copy(btab_hbm.at[idx1.at[pl.ds(0, nrows)]], dst, sem,
                             add=True),
            pltpu.async_copy(ctab_hbm.at[idxc.at[pl.ds(0, nrows)]], dst, sem,
                             add=True),
        )

    def fire_scatter(j, p):
        out, sem = bufs[p][6], bufs[p][9]
        pltpu.async_copy(out, out_hbm.at[0, pl.ds(row0 + j * _B, _B)], sem)

    def wait_scatter(p):
        out, sem = bufs[p][6], bufs[p][9]
        pltpu.make_async_copy(out, out_hbm.at[0, pl.ds(0, _B)], sem).wait()

    @pl.when(wid < 31)
    def _pipelined():
        fire_slab(0, 0)
        fire_scatter(1, 1)
        wait_slab(0)
        fire_slab(1, 1)
        compute(0)
        dense_init(0)

        def half(j_g, p, guard_slab):
            q = 1 - p
            cps = fire_gathers(p)
            wait_slab(q)

            if guard_slab:
                @pl.when(j_g + 2 < _CPW)
                def _():
                    fire_slab(j_g + 2, p)
            else:
                fire_slab(j_g + 2, p)

            compute(q)
            wait_scatter(q)
            dense_init(q)
            for cp in cps:
                cp.wait()
            fire_scatter(j_g, p)

        def pair(i, carry):
            half(2 * i, 0, False)
            half(2 * i + 1, 1, True)
            return carry

        lax.fori_loop(0, (_CPW - 1) // 2, pair, 0)

        cps = fire_gathers(0)
        for cp in cps:
            cp.wait()
        fire_scatter(_CPW - 1, 0)
        wait_scatter(1)
        wait_scatter(0)

    @pl.when(wid == 31)
    def _sequential():
        def do_chunk(base, nrows):
            slab, idx0, idx1, idxc, _, _, out, sem_in, sem_g, _ = bufs[0]
            pltpu.async_copy(feat_hbm.at[pl.ds(base, nrows)],
                             slab.at[pl.ds(0, nrows)], sem_in).wait()
            compute(0, nrows)
            dense_init(0, nrows)
            for cp in fire_gathers(0, nrows):
                cp.wait()
            pltpu.sync_copy(out.at[pl.ds(0, nrows)],
                            out_hbm.at[0, pl.ds(base, nrows)])

        def chunk(j, carry):
            do_chunk(row0 + j * _B, _B)
            return carry

        lax.fori_loop(0, _LAST_FULL, chunk, 0)
        do_chunk(jnp.int32(_TAIL_BASE), _TAIL)


def kernel(node_feat, basic_table, contact_table, W_basic, W_contact):
    wb = W_basic.reshape(_H)
    wc = W_contact.reshape(_H)
    run = pl.kernel(
        _sc_body,
        out_type=jax.ShapeDtypeStruct((1, _N, _H), jnp.float32),
        mesh=plsc.VectorSubcoreMesh(core_axis_name="c", subcore_axis_name="s"),
        compiler_params=pltpu.CompilerParams(needs_layout_passes=False),
        scratch_types=[
            pltpu.VMEM((_B, 12), jnp.float32),
            pltpu.VMEM((_B, 12), jnp.float32),
            pltpu.VMEM((_B,), jnp.int32),
            pltpu.VMEM((_B,), jnp.int32),
            pltpu.VMEM((_B,), jnp.int32),
            pltpu.VMEM((_B,), jnp.int32),
            pltpu.VMEM((_B,), jnp.int32),
            pltpu.VMEM((_B,), jnp.int32),
            pltpu.VMEM((_B,), jnp.float32),
            pltpu.VMEM((_B,), jnp.float32),
            pltpu.VMEM((_B,), jnp.float32),
            pltpu.VMEM((_B,), jnp.float32),
            pltpu.VMEM((_H,), jnp.float32),
            pltpu.VMEM((_H,), jnp.float32),
            pltpu.VMEM((_B, _H), jnp.float32),
            pltpu.VMEM((_B, _H), jnp.float32),
            pltpu.SemaphoreType.DMA,
            pltpu.SemaphoreType.DMA,
            pltpu.SemaphoreType.DMA,
            pltpu.SemaphoreType.DMA,
            pltpu.SemaphoreType.DMA,
            pltpu.SemaphoreType.DMA,
        ],
    )
    return run(node_feat, basic_table, contact_table, wb, wc)


# speedup vs baseline: 14.6733x; 1.4964x over previous
import jax
import jax.numpy as jnp
from jax import lax
from jax.experimental import pallas as pl
from jax.experimental.pallas import tpu as pltpu
from jax.experimental.pallas import tpu_sc as plsc

_N = 100000
_H = 128
_NB = 100000
_NC = 100000

_B = 128
_HB = _H // 16
_RPW = 3200
_CPW = _RPW // _B
_LAST_FULL = 6
_TAIL = 32


def _sc_body(feat_hbm, tail_hbm, btab_hbm, ctab_hbm, wb_hbm, wc_hbm, out_hbm,
             slab_a, slab_b, idx0_a, idx0_b, idx1_a, idx1_b,
             idxc_a, idxc_b, sb_a, sb_b, sc_a, sc_b,
             wb_v, wc_v, out_a, out_b,
             sem_in_a, sem_in_b, sem_g_a, sem_g_b, sem_out_a, sem_out_b):
    wid = lax.axis_index("s") * 2 + lax.axis_index("c")
    row0 = wid * _RPW

    pltpu.sync_copy(wb_hbm, wb_v)
    pltpu.sync_copy(wc_hbm, wc_v)
    wbs = [wb_v[pl.ds(h * 16, 16)] for h in range(_HB)]
    wcs = [wc_v[pl.ds(h * 16, 16)] for h in range(_HB)]

    bufs = (
        (slab_a, idx0_a, idx1_a, idxc_a, sb_a, sc_a, out_a,
         sem_in_a, sem_g_a, sem_out_a),
        (slab_b, idx0_b, idx1_b, idxc_b, sb_b, sc_b, out_b,
         sem_in_b, sem_g_b, sem_out_b),
    )

    def fire_slab(j, p):
        slab, sem = bufs[p][0], bufs[p][7]
        base = pl.multiple_of(row0 + j * _B, _B)
        pltpu.async_copy(feat_hbm.at[:, pl.ds(base, _B)], slab, sem)

    def wait_slab(p):
        slab, sem = bufs[p][0], bufs[p][7]
        pltpu.make_async_copy(feat_hbm.at[:, pl.ds(0, _B)], slab, sem).wait()

    def compute(p, nrows=_B):
        slab, idx0, idx1, idxc, sb, sc = bufs[p][:6]
        for g in range(nrows // 16):
            def col(c):
                return slab[c, pl.ds(g * 16, 16)]

            idx0[pl.ds(g * 16, 16)] = jnp.clip(
                (col(0) * _NB).astype(jnp.int32), 0, _NB - 1)
            idx1[pl.ds(g * 16, 16)] = jnp.clip(
                (col(1) * _NB).astype(jnp.int32), 0, _NB - 1)
            idxc[pl.ds(g * 16, 16)] = jnp.clip(
                (col(11) * _NC).astype(jnp.int32), 0, _NC - 1)

            s = col(2)
            for c in range(3, 10):
                s = s + col(c)
            sb[pl.ds(g * 16, 16)] = s
            sc[pl.ds(g * 16, 16)] = col(10)

    def dense_init(p, nrows=_B):
        sb, sc, out = bufs[p][4], bufs[p][5], bufs[p][6]

        def one_row(r):
            rr = jnp.full((16,), 0, jnp.int32) + r
            vb = plsc.load_gather(sb, [rr])
            vc = plsc.load_gather(sc, [rr])
            for h in range(_HB):
                out[r, pl.ds(h * 16, 16)] = vb * wbs[h] + vc * wcs[h]

        def rows2(r, _):
            one_row(2 * r)
            one_row(2 * r + 1)
            return 0

        lax.fori_loop(0, nrows // 2, rows2, 0)

    def fire_gathers(p, nrows=_B):
        idx0, idx1, idxc, out, sem = (bufs[p][1], bufs[p][2], bufs[p][3],
                                      bufs[p][6], bufs[p][8])
        dst = out.at[pl.ds(0, nrows)]
        return (
            pltpu.async_copy(btab_hbm.at[idx0.at[pl.ds(0, nrows)]], dst, sem,
                             add=True),
            pltpu.async_copy(btab_hbm.at[idx1.at[pl.ds(0, nrows)]], dst, sem,
                             add=True),
            pltpu.async_copy(ctab_hbm.at[idxc.at[pl.ds(0, nrows)]], dst, sem,
                             add=True),
        )

    def fire_scatter(j, p):
        out, sem = bufs[p][6], bufs[p][9]
        base = pl.multiple_of(row0 + j * _B, _B)
        pltpu.async_copy(out, out_hbm.at[0, pl.ds(base, _B)], sem)

    def wait_scatter(p):
        out, sem = bufs[p][6], bufs[p][9]
        pltpu.make_async_copy(out, out_hbm.at[0, pl.ds(0, _B)], sem).wait()

    @pl.when(wid < 31)
    def _pipelined():
        fire_slab(0, 0)
        fire_scatter(1, 1)
        wait_slab(0)
        fire_slab(1, 1)
        compute(0)
        dense_init(0)

        def half(j_g, p, guard_slab):
            q = 1 - p
            cps = fire_gathers(p)
            wait_slab(q)

            if guard_slab:
                @pl.when(j_g + 2 < _CPW)
                def _():
                    fire_slab(j_g + 2, p)
            else:
                fire_slab(j_g + 2, p)

            compute(q)
            wait_scatter(q)
            dense_init(q)
            for cp in cps:
                cp.wait()
            fire_scatter(j_g, p)

        def pair(i, carry):
            half(2 * i, 0, False)
            half(2 * i + 1, 1, True)
            return carry

        lax.fori_loop(0, (_CPW - 1) // 2, pair, 0)

        cps = fire_gathers(0)
        for cp in cps:
            cp.wait()
        fire_scatter(_CPW - 1, 0)
        wait_scatter(1)
        wait_scatter(0)

    @pl.when(wid == 31)
    def _sequential():
        slab, out, sem_in = bufs[0][0], bufs[0][6], bufs[0][7]

        def chunk(j, carry):
            base = pl.multiple_of(row0 + j * _B, _B)
            pltpu.async_copy(feat_hbm.at[:, pl.ds(base, _B)], slab,
                             sem_in).wait()
            compute(0)
            dense_init(0)
            for cp in fire_gathers(0):
                cp.wait()
            pltpu.sync_copy(out, out_hbm.at[0, pl.ds(base, _B)])
            return carry

        lax.fori_loop(0, _LAST_FULL, chunk, 0)

        pltpu.async_copy(tail_hbm, slab, sem_in).wait()
        compute(0)
        dense_init(0)
        for cp in fire_gathers(0):
            cp.wait()
        pltpu.sync_copy(out, out_hbm.at[0, pl.ds(_N - _B, _B)])


def kernel(node_feat, basic_table, contact_table, W_basic, W_contact):
    feat_t = node_feat.T
    tail_t = node_feat[_N - _B:].T
    wb = W_basic.reshape(_H)
    wc = W_contact.reshape(_H)
    run = pl.kernel(
        _sc_body,
        out_type=jax.ShapeDtypeStruct((1, _N, _H), jnp.float32),
        mesh=plsc.VectorSubcoreMesh(core_axis_name="c", subcore_axis_name="s"),
        compiler_params=pltpu.CompilerParams(needs_layout_passes=False),
        scratch_types=[
            pltpu.VMEM((12, _B), jnp.float32),
            pltpu.VMEM((12, _B), jnp.float32),
            pltpu.VMEM((_B,), jnp.int32),
            pltpu.VMEM((_B,), jnp.int32),
            pltpu.VMEM((_B,), jnp.int32),
            pltpu.VMEM((_B,), jnp.int32),
            pltpu.VMEM((_B,), jnp.int32),
            pltpu.VMEM((_B,), jnp.int32),
            pltpu.VMEM((_B,), jnp.float32),
            pltpu.VMEM((_B,), jnp.float32),
            pltpu.VMEM((_B,), jnp.float32),
            pltpu.VMEM((_B,), jnp.float32),
            pltpu.VMEM((_H,), jnp.float32),
            pltpu.VMEM((_H,), jnp.float32),
            pltpu.VMEM((_B, _H), jnp.float32),
            pltpu.VMEM((_B, _H), jnp.float32),
            pltpu.SemaphoreType.DMA,
            pltpu.SemaphoreType.DMA,
            pltpu.SemaphoreType.DMA,
            pltpu.SemaphoreType.DMA,
            pltpu.SemaphoreType.DMA,
            pltpu.SemaphoreType.DMA,
        ],
    )
    return run(feat_t, tail_t, basic_table, contact_table, wb, wc)
